# Initial kernel scaffold; baseline (speedup 1.0000x reference)
import jax
import jax.numpy as jnp
from jax.experimental import pallas as pl


def kernel(x, edge_index, batch, W1_l, b1_l, W1_r, W2_l, b2_l, W2_r):
    raise NotImplementedError("write your pallas kernel here")


# speedup vs baseline: 5.0269x; 5.0269x over previous
import functools

import jax
import jax.numpy as jnp
from jax import lax
from jax.experimental import pallas as pl
from jax.experimental.pallas import tpu as pltpu
from jax.experimental.pallas import tpu_sc as plsc

_N_NODES = 10000
_N_EDGES = 320000
_D_IN = 128
_D_OUT2 = 16
_N_GRAPHS = 64

_NC = 2
_NS = 16
_NW = _NC * _NS
_B = 128
_CH = 80
_EPW = _B * _CH
_PE = _EPW * _NW
_ROWS = 10240
_RPT = _ROWS // _NS


def _make_edge_agg(d, with_cnt):
  mesh = plsc.VectorSubcoreMesh(
      core_axis_name="c", subcore_axis_name="s",
      num_cores=_NC, num_subcores=_NS)
  out_type = [jax.ShapeDtypeStruct((_NC, _ROWS, d), jnp.float32)]
  scratch = [
      pltpu.VMEM((_CH, _B), jnp.int32),
      pltpu.VMEM((_CH, _B), jnp.int32),
      pltpu.VMEM((_B, d), jnp.float32),
      pltpu.VMEM_SHARED((_ROWS, d), jnp.float32),
      pltpu.SemaphoreType.DMA,
  ]
  if with_cnt:
    out_type.append(jax.ShapeDtypeStruct((_NC, _ROWS), jnp.float32))
    scratch += [
        pltpu.VMEM((_B,), jnp.float32),
        pltpu.VMEM_SHARED((_ROWS,), jnp.float32),
    ]

  def body(feat, srcs, dsts, zrows, zcnt, ones, *rest):
    if with_cnt:
      (agg_out, cnt_out, src_v, dst_v, rows_v, shared_agg, sem,
       ones_v, shared_cnt) = rest
    else:
      agg_out, src_v, dst_v, rows_v, shared_agg, sem = rest
    c = lax.axis_index("c")
    s = lax.axis_index("s")
    wid = c * _NS + s

    pltpu.sync_copy(srcs.at[wid], src_v)
    pltpu.sync_copy(dsts.at[wid], dst_v)
    pltpu.sync_copy(zrows, shared_agg.at[pl.ds(s * _RPT, _RPT)])
    if with_cnt:
      pltpu.sync_copy(zcnt, shared_cnt.at[pl.ds(s * _RPT, _RPT)])
      pltpu.sync_copy(ones, ones_v)
    plsc.subcore_barrier()

    def chunk(j, carry):
      pltpu.async_copy(feat.at[src_v.at[j]], rows_v, sem).wait()
      pltpu.sync_copy(rows_v, shared_agg.at[dst_v.at[j]], add=True)
      if with_cnt:
        pltpu.sync_copy(ones_v, shared_cnt.at[dst_v.at[j]], add=True)
      return carry

    lax.fori_loop(0, _CH, chunk, 0)
    plsc.subcore_barrier()

    pltpu.sync_copy(shared_agg.at[pl.ds(s * _RPT, _RPT)],
                    agg_out.at[c].at[pl.ds(s * _RPT, _RPT)])
    if with_cnt:
      pltpu.sync_copy(shared_cnt.at[pl.ds(s * _RPT, _RPT)],
                      cnt_out.at[c].at[pl.ds(s * _RPT, _RPT)])

  return pl.kernel(
      body, out_type=out_type, mesh=mesh, scratch_types=scratch,
      compiler_params=pltpu.CompilerParams(use_tc_tiling_on_sc=False))


def _tc1_body(x_ref, agg_ref, cnt_ref, w1l_ref, b1_ref, w1r_ref,
              w2l_ref, b2_ref, w2r_ref, y2_ref, r2_ref):
  cnt = jnp.maximum(cnt_ref[0] + cnt_ref[1], 1.0)
  mean = (agg_ref[0] + agg_ref[1]) / cnt
  h = jnp.dot(mean, w1l_ref[...], preferred_element_type=jnp.float32)
  h = h + b1_ref[...]
  h = h + jnp.dot(x_ref[...], w1r_ref[...], preferred_element_type=jnp.float32)
  h = jnp.maximum(h, 0.0)
  y2_ref[...] = jnp.dot(h, w2l_ref[...], preferred_element_type=jnp.float32)
  r2_ref[...] = (jnp.dot(h, w2r_ref[...], preferred_element_type=jnp.float32)
                 + b2_ref[...])


def _tc2_body(agg2_ref, cnt_ref, r2_ref, batch_ref, out_ref):
  cnt = jnp.maximum(cnt_ref[0] + cnt_ref[1], 1.0)
  h2 = (agg2_ref[0] + agg2_ref[1]) / cnt + r2_ref[...]
  onehot = (lax.broadcasted_iota(jnp.int32, (_N_GRAPHS, _ROWS), 0)
            == batch_ref[...]).astype(jnp.float32)
  psum = jnp.dot(onehot, h2, preferred_element_type=jnp.float32)
  gcnt = jnp.sum(onehot, axis=1, keepdims=True)
  pooled = psum / jnp.maximum(gcnt, 1.0)
  m = jnp.max(pooled, axis=1, keepdims=True)
  lse = m + jnp.log(jnp.sum(jnp.exp(pooled - m), axis=1, keepdims=True))
  out_ref[...] = pooled - lse


def kernel(x, edge_index, batch, W1_l, b1_l, W1_r, W2_l, b2_l, W2_r):
  src = edge_index[0]
  dst = edge_index[1]
  pad = _PE - _N_EDGES
  src_p = jnp.concatenate(
      [src, jnp.zeros((pad,), jnp.int32)]).reshape(_NW, _CH, _B)
  dst_p = jnp.concatenate(
      [dst, jnp.full((pad,), _N_NODES, jnp.int32)]).reshape(_NW, _CH, _B)
  x_p = jnp.pad(x, ((0, _ROWS - _N_NODES), (0, 0)))
  batch_p = jnp.concatenate(
      [batch, jnp.full((_ROWS - _N_NODES,), _N_GRAPHS, jnp.int32)]
  ).reshape(1, _ROWS)
  z128 = jnp.zeros((_RPT, _D_IN), jnp.float32)
  z16 = jnp.zeros((_RPT, _D_OUT2), jnp.float32)
  zc = jnp.zeros((_RPT,), jnp.float32)
  ones = jnp.ones((_B,), jnp.float32)

  agg1, cnt = _make_edge_agg(_D_IN, True)(x_p, src_p, dst_p, z128, zc, ones)
  cnt3 = cnt.reshape(_NC, _ROWS, 1)

  y2, r2 = pl.pallas_call(
      _tc1_body,
      out_shape=[jax.ShapeDtypeStruct((_ROWS, _D_OUT2), jnp.float32),
                 jax.ShapeDtypeStruct((_ROWS, _D_OUT2), jnp.float32)],
  )(x_p, agg1, cnt3, W1_l, b1_l.reshape(1, -1), W1_r,
    W2_l, b2_l.reshape(1, -1), W2_r)

  (agg2,) = _make_edge_agg(_D_OUT2, False)(y2, src_p, dst_p, z16, zc, ones)

  out = pl.pallas_call(
      _tc2_body,
      out_shape=jax.ShapeDtypeStruct((_N_GRAPHS, _D_OUT2), jnp.float32),
  )(agg2, cnt3, r2, batch_p)
  return out


# speedup vs baseline: 5.8564x; 1.1650x over previous
import functools

import jax
import jax.numpy as jnp
from jax import lax
from jax.experimental import pallas as pl
from jax.experimental.pallas import tpu as pltpu
from jax.experimental.pallas import tpu_sc as plsc

_N_NODES = 10000
_N_EDGES = 320000
_D_IN = 128
_D_OUT2 = 16
_N_GRAPHS = 64

_NC = 2
_NS = 16
_NW = _NC * _NS
_B1, _CH1 = 64, 160
_B2, _CH2 = 128, 80
_EPW = _B1 * _CH1
_PE = _EPW * _NW
_ROWS = 10240
_RPT = _ROWS // _NS


def _make_edge_agg(d, with_cnt, bs, ch):
  mesh = plsc.VectorSubcoreMesh(
      core_axis_name="c", subcore_axis_name="s",
      num_cores=_NC, num_subcores=_NS)
  out_type = [jax.ShapeDtypeStruct((_NC, _ROWS, d), jnp.float32)]
  scratch = [
      pltpu.VMEM((ch, bs), jnp.int32),
      pltpu.VMEM((ch, bs), jnp.int32),
      pltpu.VMEM((2, bs, d), jnp.float32),
      pltpu.VMEM_SHARED((_ROWS, d), jnp.float32),
      pltpu.SemaphoreType.DMA,
      pltpu.SemaphoreType.DMA,
      pltpu.SemaphoreType.DMA,
      pltpu.SemaphoreType.DMA,
  ]
  if with_cnt:
    out_type.append(jax.ShapeDtypeStruct((_NC, _ROWS), jnp.float32))
    scratch += [
        pltpu.VMEM((bs,), jnp.float32),
        pltpu.VMEM_SHARED((_ROWS,), jnp.float32),
    ]

  def body(feat, srcs, dsts, zrows, zcnt, ones, *rest):
    if with_cnt:
      (agg_out, cnt_out, src_v, dst_v, rows_v, shared_agg,
       g0, g1, s0, s1, ones_v, shared_cnt) = rest
    else:
      (agg_out, src_v, dst_v, rows_v, shared_agg, g0, g1, s0, s1) = rest
    gsem = (g0, g1)
    ssem = (s0, s1)
    c = lax.axis_index("c")
    s = lax.axis_index("s")
    wid = c * _NS + s

    pltpu.sync_copy(srcs.at[wid], src_v)
    pltpu.sync_copy(dsts.at[wid], dst_v)
    pltpu.sync_copy(zrows, shared_agg.at[pl.ds(s * _RPT, _RPT)])
    if with_cnt:
      pltpu.sync_copy(zcnt, shared_cnt.at[pl.ds(s * _RPT, _RPT)])
      pltpu.sync_copy(ones, ones_v)
    plsc.subcore_barrier()

    for b in range(2):
      pltpu.async_copy(feat.at[src_v.at[b]], rows_v.at[b], gsem[b])

    def step(jj, carry):
      for b in range(2):
        j = jj * 2 + b
        pltpu.make_async_copy(
            feat.at[src_v.at[j]], rows_v.at[b], gsem[b]).wait()
        h = pltpu.async_copy(
            rows_v.at[b], shared_agg.at[dst_v.at[j]], ssem[b], add=True)
        if with_cnt:
          h2 = pltpu.async_copy(
              ones_v, shared_cnt.at[dst_v.at[j]], ssem[b], add=True)
        h.wait()
        if with_cnt:
          h2.wait()

        @pl.when(jj < ch // 2 - 1)
        def _():
          pltpu.async_copy(feat.at[src_v.at[j + 2]], rows_v.at[b], gsem[b])
      return carry

    lax.fori_loop(0, ch // 2, step, 0)
    plsc.subcore_barrier()

    pltpu.sync_copy(shared_agg.at[pl.ds(s * _RPT, _RPT)],
                    agg_out.at[c].at[pl.ds(s * _RPT, _RPT)])
    if with_cnt:
      pltpu.sync_copy(shared_cnt.at[pl.ds(s * _RPT, _RPT)],
                      cnt_out.at[c].at[pl.ds(s * _RPT, _RPT)])

  return pl.kernel(
      body, out_type=out_type, mesh=mesh, scratch_types=scratch,
      compiler_params=pltpu.CompilerParams(use_tc_tiling_on_sc=False))


def _tc1_body(x_ref, agg_ref, cnt_ref, w1l_ref, b1_ref, w1r_ref,
              w2l_ref, b2_ref, w2r_ref, y2_ref, r2_ref):
  cnt = jnp.maximum(cnt_ref[0] + cnt_ref[1], 1.0)
  mean = (agg_ref[0] + agg_ref[1]) / cnt
  h = jnp.dot(mean, w1l_ref[...], preferred_element_type=jnp.float32)
  h = h + b1_ref[...]
  h = h + jnp.dot(x_ref[...], w1r_ref[...], preferred_element_type=jnp.float32)
  h = jnp.maximum(h, 0.0)
  y2_ref[...] = jnp.dot(h, w2l_ref[...], preferred_element_type=jnp.float32)
  r2_ref[...] = (jnp.dot(h, w2r_ref[...], preferred_element_type=jnp.float32)
                 + b2_ref[...])


def _tc2_body(agg2_ref, cnt_ref, r2_ref, batch_ref, out_ref):
  cnt = jnp.maximum(cnt_ref[0] + cnt_ref[1], 1.0)
  h2 = (agg2_ref[0] + agg2_ref[1]) / cnt + r2_ref[...]
  onehot = (lax.broadcasted_iota(jnp.int32, (_N_GRAPHS, _ROWS), 0)
            == batch_ref[...]).astype(jnp.float32)
  psum = jnp.dot(onehot, h2, preferred_element_type=jnp.float32)
  gcnt = jnp.sum(onehot, axis=1, keepdims=True)
  pooled = psum / jnp.maximum(gcnt, 1.0)
  m = jnp.max(pooled, axis=1, keepdims=True)
  lse = m + jnp.log(jnp.sum(jnp.exp(pooled - m), axis=1, keepdims=True))
  out_ref[...] = pooled - lse


def kernel(x, edge_index, batch, W1_l, b1_l, W1_r, W2_l, b2_l, W2_r):
  src = edge_index[0]
  dst = edge_index[1]
  pad = _PE - _N_EDGES
  src_f = jnp.concatenate([src, jnp.zeros((pad,), jnp.int32)])
  dst_f = jnp.concatenate([dst, jnp.full((pad,), _N_NODES, jnp.int32)])
  x_p = jnp.pad(x, ((0, _ROWS - _N_NODES), (0, 0)))
  batch_p = jnp.concatenate(
      [batch, jnp.full((_ROWS - _N_NODES,), _N_GRAPHS, jnp.int32)]
  ).reshape(1, _ROWS)
  z128 = jnp.zeros((_RPT, _D_IN), jnp.float32)
  z16 = jnp.zeros((_RPT, _D_OUT2), jnp.float32)
  zc = jnp.zeros((_RPT,), jnp.float32)
  ones1 = jnp.ones((_B1,), jnp.float32)

  agg1, cnt = _make_edge_agg(_D_IN, True, _B1, _CH1)(
      x_p, src_f.reshape(_NW, _CH1, _B1), dst_f.reshape(_NW, _CH1, _B1),
      z128, zc, ones1)
  cnt3 = cnt.reshape(_NC, _ROWS, 1)

  y2, r2 = pl.pallas_call(
      _tc1_body,
      out_shape=[jax.ShapeDtypeStruct((_ROWS, _D_OUT2), jnp.float32),
                 jax.ShapeDtypeStruct((_ROWS, _D_OUT2), jnp.float32)],
  )(x_p, agg1, cnt3, W1_l, b1_l.reshape(1, -1), W1_r,
    W2_l, b2_l.reshape(1, -1), W2_r)

  (agg2,) = _make_edge_agg(_D_OUT2, False, _B2, _CH2)(
      y2, src_f.reshape(_NW, _CH2, _B2), dst_f.reshape(_NW, _CH2, _B2),
      z16, zc, ones1)

  out = pl.pallas_call(
      _tc2_body,
      out_shape=jax.ShapeDtypeStruct((_N_GRAPHS, _D_OUT2), jnp.float32),
  )(agg2, cnt3, r2, batch_p)
  return out


# speedup vs baseline: 13.8637x; 2.3673x over previous
import functools

import jax
import jax.numpy as jnp
from jax import lax
from jax.experimental import pallas as pl
from jax.experimental.pallas import tpu as pltpu
from jax.experimental.pallas import tpu_sc as plsc

_N_NODES = 10000
_N_EDGES = 320000
_D_IN = 128
_D_OUT2 = 16
_N_GRAPHS = 64

_NC = 2
_NS = 16
_NW = _NC * _NS
_BS, _CH = 100, 100
_ROWS = 10240
_RPT = _ROWS // _NS


def _make_edge_agg(d, with_cnt, bs, ch):
  mesh = plsc.VectorSubcoreMesh(
      core_axis_name="c", subcore_axis_name="s",
      num_cores=_NC, num_subcores=_NS)
  out_type = [jax.ShapeDtypeStruct((_NC, _ROWS, d), jnp.float32)]
  scratch = [
      pltpu.VMEM((ch, bs), jnp.int32),
      pltpu.VMEM((ch, bs), jnp.int32),
      pltpu.VMEM((2, bs, d), jnp.float32),
      pltpu.VMEM_SHARED((_ROWS, d), jnp.float32),
      pltpu.SemaphoreType.DMA,
      pltpu.SemaphoreType.DMA,
      pltpu.SemaphoreType.DMA,
      pltpu.SemaphoreType.DMA,
  ]
  if with_cnt:
    out_type.append(jax.ShapeDtypeStruct((_NC, _ROWS), jnp.float32))
    scratch += [
        pltpu.VMEM((bs,), jnp.float32),
        pltpu.VMEM_SHARED((_ROWS,), jnp.float32),
    ]

  def body(feat, srcs, dsts, zrows, zcnt, ones, *rest):
    if with_cnt:
      (agg_out, cnt_out, src_v, dst_v, rows_v, shared_agg,
       g0, g1, s0, s1, ones_v, shared_cnt) = rest
    else:
      (agg_out, src_v, dst_v, rows_v, shared_agg, g0, g1, s0, s1) = rest
    gsem = (g0, g1)
    ssem = (s0, s1)
    c = lax.axis_index("c")
    s = lax.axis_index("s")
    wid = c * _NS + s

    pltpu.sync_copy(srcs.at[wid], src_v)
    pltpu.sync_copy(dsts.at[wid], dst_v)
    pltpu.sync_copy(zrows, shared_agg.at[pl.ds(s * _RPT, _RPT)])
    if with_cnt:
      pltpu.sync_copy(zcnt, shared_cnt.at[pl.ds(s * _RPT, _RPT)])
      pltpu.sync_copy(ones, ones_v)
    plsc.subcore_barrier()

    for b in range(2):
      pltpu.async_copy(feat.at[src_v.at[b]], rows_v.at[b], gsem[b])

    def step(jj, carry):
      for b in range(2):
        j = jj * 2 + b
        pltpu.make_async_copy(
            feat.at[src_v.at[j]], rows_v.at[b], gsem[b]).wait()
        h = pltpu.async_copy(
            rows_v.at[b], shared_agg.at[dst_v.at[j]], ssem[b], add=True)
        if with_cnt:
          h2 = pltpu.async_copy(
              ones_v, shared_cnt.at[dst_v.at[j]], ssem[b], add=True)
        h.wait()
        if with_cnt:
          h2.wait()

        @pl.when(jj < ch // 2 - 1)
        def _():
          pltpu.async_copy(feat.at[src_v.at[j + 2]], rows_v.at[b], gsem[b])
      return carry

    lax.fori_loop(0, ch // 2, step, 0)
    plsc.subcore_barrier()

    pltpu.sync_copy(shared_agg.at[pl.ds(s * _RPT, _RPT)],
                    agg_out.at[c].at[pl.ds(s * _RPT, _RPT)])
    if with_cnt:
      pltpu.sync_copy(shared_cnt.at[pl.ds(s * _RPT, _RPT)],
                      cnt_out.at[c].at[pl.ds(s * _RPT, _RPT)])

  return pl.kernel(
      body, out_type=out_type, mesh=mesh, scratch_types=scratch,
      compiler_params=pltpu.CompilerParams(use_tc_tiling_on_sc=False))


def _tc1_body(x_ref, agg_ref, cnt_ref, w1l_ref, b1_ref, w1r_ref,
              w2l_ref, b2_ref, w2r_ref, y2_ref, r2_ref):
  cnt = jnp.maximum(cnt_ref[0] + cnt_ref[1], 1.0)
  mean = (agg_ref[0] + agg_ref[1]) / cnt
  h = jnp.dot(mean, w1l_ref[...], preferred_element_type=jnp.float32)
  h = h + b1_ref[...]
  h = h + jnp.dot(x_ref[...], w1r_ref[...], preferred_element_type=jnp.float32)
  h = jnp.maximum(h, 0.0)
  y2_ref[...] = jnp.dot(h, w2l_ref[...], preferred_element_type=jnp.float32)
  r2_ref[...] = (jnp.dot(h, w2r_ref[...], preferred_element_type=jnp.float32)
                 + b2_ref[...])


def _tc2_body(agg2_ref, cnt_ref, r2_ref, batch_ref, out_ref):
  cnt = jnp.maximum(cnt_ref[0] + cnt_ref[1], 1.0)
  h2 = (agg2_ref[0] + agg2_ref[1]) / cnt + r2_ref[...]
  onehot = (lax.broadcasted_iota(jnp.int32, (_N_GRAPHS, _ROWS), 0)
            == batch_ref[...]).astype(jnp.float32)
  psum = jnp.dot(onehot, h2, preferred_element_type=jnp.float32)
  gcnt = jnp.sum(onehot, axis=1, keepdims=True)
  pooled = psum / jnp.maximum(gcnt, 1.0)
  m = jnp.max(pooled, axis=1, keepdims=True)
  lse = m + jnp.log(jnp.sum(jnp.exp(pooled - m), axis=1, keepdims=True))
  out_ref[...] = pooled - lse


def kernel(x, edge_index, batch, W1_l, b1_l, W1_r, W2_l, b2_l, W2_r):
  src_r = edge_index[0].reshape(_NW, _CH, _BS)
  dst_r = edge_index[1].reshape(_NW, _CH, _BS)
  x_p = jnp.pad(x, ((0, _ROWS - _N_NODES), (0, 0)))
  batch_p = jnp.concatenate(
      [batch, jnp.full((_ROWS - _N_NODES,), _N_GRAPHS, jnp.int32)]
  ).reshape(1, _ROWS)
  z128 = jnp.zeros((_RPT, _D_IN), jnp.float32)
  z16 = jnp.zeros((_RPT, _D_OUT2), jnp.float32)
  zc = jnp.zeros((_RPT,), jnp.float32)
  ones1 = jnp.ones((_BS,), jnp.float32)

  agg1, cnt = _make_edge_agg(_D_IN, True, _BS, _CH)(
      x_p, src_r, dst_r, z128, zc, ones1)
  cnt3 = cnt.reshape(_NC, _ROWS, 1)

  y2, r2 = pl.pallas_call(
      _tc1_body,
      out_shape=[jax.ShapeDtypeStruct((_ROWS, _D_OUT2), jnp.float32),
                 jax.ShapeDtypeStruct((_ROWS, _D_OUT2), jnp.float32)],
  )(x_p, agg1, cnt3, W1_l, b1_l.reshape(1, -1), W1_r,
    W2_l, b2_l.reshape(1, -1), W2_r)

  (agg2,) = _make_edge_agg(_D_OUT2, False, _BS, _CH)(
      y2, src_r, dst_r, z16, zc, ones1)

  out = pl.pallas_call(
      _tc2_body,
      out_shape=jax.ShapeDtypeStruct((_N_GRAPHS, _D_OUT2), jnp.float32),
  )(agg2, cnt3, r2, batch_p)
  return out


# speedup vs baseline: 15.9299x; 1.1490x over previous
import functools

import jax
import jax.numpy as jnp
from jax import lax
from jax.experimental import pallas as pl
from jax.experimental.pallas import tpu as pltpu
from jax.experimental.pallas import tpu_sc as plsc

_N_NODES = 10000
_N_EDGES = 320000
_D_IN = 128
_D_OUT2 = 16
_N_GRAPHS = 64

_NC = 2
_NS = 16
_NW = _NC * _NS
_B1, _C1, _NB1 = 50, 200, 4
_B2, _C2, _NB2 = 100, 100, 4
_ROWS = 10240
_RPT = _ROWS // _NS


def _make_edge_agg(d, with_cnt, bs, ch, nbuf):
  mesh = plsc.VectorSubcoreMesh(
      core_axis_name="c", subcore_axis_name="s",
      num_cores=_NC, num_subcores=_NS)
  out_type = [jax.ShapeDtypeStruct((_NC, _ROWS, d), jnp.float32)]
  scratch = [
      pltpu.VMEM((ch, bs), jnp.int32),
      pltpu.VMEM((ch, bs), jnp.int32),
      pltpu.VMEM((nbuf, bs, d), jnp.float32),
      pltpu.VMEM_SHARED((_ROWS, d), jnp.float32),
  ]
  scratch += [pltpu.SemaphoreType.DMA] * (2 * nbuf)
  if with_cnt:
    out_type.append(jax.ShapeDtypeStruct((_NC, _ROWS), jnp.float32))
    scratch += [
        pltpu.VMEM((bs,), jnp.float32),
        pltpu.VMEM_SHARED((_ROWS,), jnp.float32),
    ]

  def body(feat, srcs, dsts, zrows, zcnt, ones, *rest):
    if with_cnt:
      (agg_out, cnt_out, src_v, dst_v, rows_v, shared_agg,
       *sems, ones_v, shared_cnt) = rest
    else:
      (agg_out, src_v, dst_v, rows_v, shared_agg, *sems) = rest
    gsem = sems[:nbuf]
    ssem = sems[nbuf:]
    c = lax.axis_index("c")
    s = lax.axis_index("s")
    wid = c * _NS + s

    pltpu.sync_copy(srcs.at[wid], src_v)
    pltpu.sync_copy(dsts.at[wid], dst_v)
    pltpu.sync_copy(zrows, shared_agg.at[pl.ds(s * _RPT, _RPT)])
    if with_cnt:
      pltpu.sync_copy(zcnt, shared_cnt.at[pl.ds(s * _RPT, _RPT)])
      pltpu.sync_copy(ones, ones_v)
    plsc.subcore_barrier()

    for b in range(nbuf):
      pltpu.async_copy(feat.at[src_v.at[b]], rows_v.at[b], gsem[b])

    def step(jj, carry):
      for b in range(nbuf):
        j = jj * nbuf + b
        pltpu.make_async_copy(
            feat.at[src_v.at[j]], rows_v.at[b], gsem[b]).wait()
        h = pltpu.async_copy(
            rows_v.at[b], shared_agg.at[dst_v.at[j]], ssem[b], add=True)
        if with_cnt:
          h2 = pltpu.async_copy(
              ones_v, shared_cnt.at[dst_v.at[j]], ssem[b], add=True)
        h.wait()
        if with_cnt:
          h2.wait()

        @pl.when(jj < ch // nbuf - 1)
        def _():
          pltpu.async_copy(
              feat.at[src_v.at[j + nbuf]], rows_v.at[b], gsem[b])
      return carry

    lax.fori_loop(0, ch // nbuf, step, 0)
    plsc.subcore_barrier()

    pltpu.sync_copy(shared_agg.at[pl.ds(s * _RPT, _RPT)],
                    agg_out.at[c].at[pl.ds(s * _RPT, _RPT)])
    if with_cnt:
      pltpu.sync_copy(shared_cnt.at[pl.ds(s * _RPT, _RPT)],
                      cnt_out.at[c].at[pl.ds(s * _RPT, _RPT)])

  return pl.kernel(
      body, out_type=out_type, mesh=mesh, scratch_types=scratch,
      compiler_params=pltpu.CompilerParams(use_tc_tiling_on_sc=False))


def _tc1_body(x_ref, agg_ref, cnt_ref, w1l_ref, b1_ref, w1r_ref,
              w2l_ref, b2_ref, w2r_ref, y2_ref, r2_ref):
  cnt = jnp.maximum(cnt_ref[0] + cnt_ref[1], 1.0)
  mean = (agg_ref[0] + agg_ref[1]) / cnt
  h = jnp.dot(mean, w1l_ref[...], preferred_element_type=jnp.float32)
  h = h + b1_ref[...]
  h = h + jnp.dot(x_ref[...], w1r_ref[...], preferred_element_type=jnp.float32)
  h = jnp.maximum(h, 0.0)
  y2_ref[...] = jnp.dot(h, w2l_ref[...], preferred_element_type=jnp.float32)
  r2_ref[...] = (jnp.dot(h, w2r_ref[...], preferred_element_type=jnp.float32)
                 + b2_ref[...])


def _tc2_body(agg2_ref, cnt_ref, r2_ref, batch_ref, out_ref):
  cnt = jnp.maximum(cnt_ref[0] + cnt_ref[1], 1.0)
  h2 = (agg2_ref[0] + agg2_ref[1]) / cnt + r2_ref[...]
  onehot = (lax.broadcasted_iota(jnp.int32, (_N_GRAPHS, _ROWS), 0)
            == batch_ref[...]).astype(jnp.float32)
  psum = jnp.dot(onehot, h2, preferred_element_type=jnp.float32)
  gcnt = jnp.sum(onehot, axis=1, keepdims=True)
  pooled = psum / jnp.maximum(gcnt, 1.0)
  m = jnp.max(pooled, axis=1, keepdims=True)
  lse = m + jnp.log(jnp.sum(jnp.exp(pooled - m), axis=1, keepdims=True))
  out_ref[...] = pooled - lse


def kernel(x, edge_index, batch, W1_l, b1_l, W1_r, W2_l, b2_l, W2_r):
  src_r1 = edge_index[0].reshape(_NW, _C1, _B1)
  dst_r1 = edge_index[1].reshape(_NW, _C1, _B1)
  src_r2 = edge_index[0].reshape(_NW, _C2, _B2)
  dst_r2 = edge_index[1].reshape(_NW, _C2, _B2)
  x_p = jnp.pad(x, ((0, _ROWS - _N_NODES), (0, 0)))
  batch_p = jnp.concatenate(
      [batch, jnp.full((_ROWS - _N_NODES,), _N_GRAPHS, jnp.int32)]
  ).reshape(1, _ROWS)
  z128 = jnp.zeros((_RPT, _D_IN), jnp.float32)
  z16 = jnp.zeros((_RPT, _D_OUT2), jnp.float32)
  zc = jnp.zeros((_RPT,), jnp.float32)
  ones1 = jnp.ones((_B1,), jnp.float32)

  agg1, cnt = _make_edge_agg(_D_IN, True, _B1, _C1, _NB1)(
      x_p, src_r1, dst_r1, z128, zc, ones1)
  cnt3 = cnt.reshape(_NC, _ROWS, 1)

  y2, r2 = pl.pallas_call(
      _tc1_body,
      out_shape=[jax.ShapeDtypeStruct((_ROWS, _D_OUT2), jnp.float32),
                 jax.ShapeDtypeStruct((_ROWS, _D_OUT2), jnp.float32)],
  )(x_p, agg1, cnt3, W1_l, b1_l.reshape(1, -1), W1_r,
    W2_l, b2_l.reshape(1, -1), W2_r)

  (agg2,) = _make_edge_agg(_D_OUT2, False, _B2, _C2, _NB2)(
      y2, src_r2, dst_r2, z16, zc, ones1)

  out = pl.pallas_call(
      _tc2_body,
      out_shape=jax.ShapeDtypeStruct((_N_GRAPHS, _D_OUT2), jnp.float32),
  )(agg2, cnt3, r2, batch_p)
  return out


# speedup vs baseline: 16.2439x; 1.0197x over previous
import functools

import jax
import jax.numpy as jnp
from jax import lax
from jax.experimental import pallas as pl
from jax.experimental.pallas import tpu as pltpu
from jax.experimental.pallas import tpu_sc as plsc

_N_NODES = 10000
_N_EDGES = 320000
_D_IN = 128
_D_OUT2 = 16
_N_GRAPHS = 64

_NC = 2
_NS = 16
_NW = _NC * _NS
_B1, _C1, _NB1 = 50, 200, 4
_B2, _C2, _NB2 = 100, 100, 4
_ROWS = 10112
_RPT = _ROWS // _NS


def _make_edge_agg(d, with_cnt, bs, ch, nbuf):
  mesh = plsc.VectorSubcoreMesh(
      core_axis_name="c", subcore_axis_name="s",
      num_cores=_NC, num_subcores=_NS)
  out_type = [jax.ShapeDtypeStruct((_NC, _ROWS, d), jnp.float32)]
  scratch = [
      pltpu.VMEM((ch, bs), jnp.int32),
      pltpu.VMEM((ch, bs), jnp.int32),
      pltpu.VMEM((nbuf, bs, d), jnp.float32),
      pltpu.VMEM_SHARED((_ROWS, d), jnp.float32),
  ]
  scratch += [pltpu.SemaphoreType.DMA] * (2 * nbuf)
  if with_cnt:
    out_type.append(jax.ShapeDtypeStruct((_NC, _ROWS), jnp.float32))
    scratch += [
        pltpu.VMEM((bs,), jnp.float32),
        pltpu.VMEM_SHARED((_ROWS,), jnp.float32),
    ]

  def body(feat, srcs, dsts, zrows, zcnt, ones, *rest):
    if with_cnt:
      (agg_out, cnt_out, src_v, dst_v, rows_v, shared_agg,
       *sems, ones_v, shared_cnt) = rest
    else:
      (agg_out, src_v, dst_v, rows_v, shared_agg, *sems) = rest
    gsem = sems[:nbuf]
    ssem = sems[nbuf:]
    c = lax.axis_index("c")
    s = lax.axis_index("s")
    wid = c * _NS + s

    pltpu.sync_copy(srcs.at[wid], src_v)
    pltpu.sync_copy(dsts.at[wid], dst_v)
    pltpu.sync_copy(zrows, shared_agg.at[pl.ds(s * _RPT, _RPT)])
    if with_cnt:
      pltpu.sync_copy(zcnt, shared_cnt.at[pl.ds(s * _RPT, _RPT)])
      pltpu.sync_copy(ones, ones_v)
    plsc.subcore_barrier()

    for b in range(nbuf):
      pltpu.async_copy(feat.at[src_v.at[b]], rows_v.at[b], gsem[b])

    def step(jj, carry):
      for b in range(nbuf):
        j = jj * nbuf + b
        pltpu.make_async_copy(
            feat.at[src_v.at[j]], rows_v.at[b], gsem[b]).wait()
        h = pltpu.async_copy(
            rows_v.at[b], shared_agg.at[dst_v.at[j]], ssem[b], add=True)
        if with_cnt:
          h2 = pltpu.async_copy(
              ones_v, shared_cnt.at[dst_v.at[j]], ssem[b], add=True)
        h.wait()
        if with_cnt:
          h2.wait()

        @pl.when(jj < ch // nbuf - 1)
        def _():
          pltpu.async_copy(
              feat.at[src_v.at[j + nbuf]], rows_v.at[b], gsem[b])
      return carry

    lax.fori_loop(0, ch // nbuf, step, 0)
    plsc.subcore_barrier()

    pltpu.sync_copy(shared_agg.at[pl.ds(s * _RPT, _RPT)],
                    agg_out.at[c].at[pl.ds(s * _RPT, _RPT)])
    if with_cnt:
      pltpu.sync_copy(shared_cnt.at[pl.ds(s * _RPT, _RPT)],
                      cnt_out.at[c].at[pl.ds(s * _RPT, _RPT)])

  return pl.kernel(
      body, out_type=out_type, mesh=mesh, scratch_types=scratch,
      compiler_params=pltpu.CompilerParams(use_tc_tiling_on_sc=False))


def _tc1_body(x_ref, agg_ref, cnt_ref, w1l_ref, b1_ref, w1r_ref,
              w2l_ref, b2_ref, w2r_ref, y2_ref, r2_ref):
  cnt = jnp.maximum(cnt_ref[0, :_N_NODES] + cnt_ref[1, :_N_NODES], 1.0)
  mean = (agg_ref[0, :_N_NODES] + agg_ref[1, :_N_NODES]) / cnt
  h = jnp.dot(mean, w1l_ref[...], preferred_element_type=jnp.float32)
  h = h + b1_ref[...]
  h = h + jnp.dot(x_ref[...], w1r_ref[...], preferred_element_type=jnp.float32)
  h = jnp.maximum(h, 0.0)
  y2_ref[...] = jnp.dot(h, w2l_ref[...], preferred_element_type=jnp.float32)
  r2_ref[...] = (jnp.dot(h, w2r_ref[...], preferred_element_type=jnp.float32)
                 + b2_ref[...])


def _tc2_body(agg2_ref, cnt_ref, r2_ref, batch_ref, out_ref):
  cnt = jnp.maximum(cnt_ref[0, :_N_NODES] + cnt_ref[1, :_N_NODES], 1.0)
  h2 = (agg2_ref[0, :_N_NODES] + agg2_ref[1, :_N_NODES]) / cnt + r2_ref[...]
  onehot = (lax.broadcasted_iota(jnp.int32, (_N_GRAPHS, _N_NODES), 0)
            == batch_ref[...]).astype(jnp.float32)
  psum = jnp.dot(onehot, h2, preferred_element_type=jnp.float32)
  gcnt = jnp.sum(onehot, axis=1, keepdims=True)
  pooled = psum / jnp.maximum(gcnt, 1.0)
  m = jnp.max(pooled, axis=1, keepdims=True)
  lse = m + jnp.log(jnp.sum(jnp.exp(pooled - m), axis=1, keepdims=True))
  out_ref[...] = pooled - lse


def kernel(x, edge_index, batch, W1_l, b1_l, W1_r, W2_l, b2_l, W2_r):
  src_r1 = edge_index[0].reshape(_NW, _C1, _B1)
  dst_r1 = edge_index[1].reshape(_NW, _C1, _B1)
  src_r2 = edge_index[0].reshape(_NW, _C2, _B2)
  dst_r2 = edge_index[1].reshape(_NW, _C2, _B2)
  batch_p = batch.reshape(1, _N_NODES)
  z128 = jnp.zeros((_RPT, _D_IN), jnp.float32)
  z16 = jnp.zeros((_RPT, _D_OUT2), jnp.float32)
  zc = jnp.zeros((_RPT,), jnp.float32)
  ones1 = jnp.ones((_B1,), jnp.float32)

  agg1, cnt = _make_edge_agg(_D_IN, True, _B1, _C1, _NB1)(
      x, src_r1, dst_r1, z128, zc, ones1)
  cnt3 = cnt.reshape(_NC, _ROWS, 1)

  y2, r2 = pl.pallas_call(
      _tc1_body,
      out_shape=[jax.ShapeDtypeStruct((_N_NODES, _D_OUT2), jnp.float32),
                 jax.ShapeDtypeStruct((_N_NODES, _D_OUT2), jnp.float32)],
  )(x, agg1, cnt3, W1_l, b1_l.reshape(1, -1), W1_r,
    W2_l, b2_l.reshape(1, -1), W2_r)

  (agg2,) = _make_edge_agg(_D_OUT2, False, _B2, _C2, _NB2)(
      y2, src_r2, dst_r2, z16, zc, ones1)

  out = pl.pallas_call(
      _tc2_body,
      out_shape=jax.ShapeDtypeStruct((_N_GRAPHS, _D_OUT2), jnp.float32),
  )(agg2, cnt3, r2, batch_p)
  return out


# speedup vs baseline: 17.1714x; 1.0571x over previous
import functools

import jax
import jax.numpy as jnp
from jax import lax
from jax.experimental import pallas as pl
from jax.experimental.pallas import tpu as pltpu
from jax.experimental.pallas import tpu_sc as plsc

_N_NODES = 10000
_N_EDGES = 320000
_D_IN = 128
_D_OUT2 = 16
_N_GRAPHS = 64

_NC = 2
_NS = 16
_NW = _NC * _NS
_BS, _CH, _NB = 50, 200, 4
_ROWS = 10112
_RPT = _ROWS // _NS


def _make_edge_agg(d, with_cnt, bs, ch, nbuf):
  mesh = plsc.VectorSubcoreMesh(
      core_axis_name="c", subcore_axis_name="s",
      num_cores=_NC, num_subcores=_NS)
  out_type = [jax.ShapeDtypeStruct((_NC, _ROWS, d), jnp.float32)]
  scratch = [
      pltpu.VMEM((ch, bs), jnp.int32),
      pltpu.VMEM((ch, bs), jnp.int32),
      pltpu.VMEM((nbuf, bs, d), jnp.float32),
      pltpu.VMEM_SHARED((_ROWS, d), jnp.float32),
  ]
  scratch += [pltpu.SemaphoreType.DMA] * (2 * nbuf)
  if with_cnt:
    out_type.append(jax.ShapeDtypeStruct((_NC, _ROWS), jnp.float32))
    scratch += [
        pltpu.VMEM((bs,), jnp.float32),
        pltpu.VMEM_SHARED((_ROWS,), jnp.float32),
    ]

  def body(feat, edges, zrows, zcnt, ones, *rest):
    if with_cnt:
      (agg_out, cnt_out, src_v, dst_v, rows_v, shared_agg,
       *sems, ones_v, shared_cnt) = rest
    else:
      (agg_out, src_v, dst_v, rows_v, shared_agg, *sems) = rest
    gsem = sems[:nbuf]
    ssem = sems[nbuf:]
    c = lax.axis_index("c")
    s = lax.axis_index("s")
    wid = c * _NS + s

    pltpu.sync_copy(edges.at[0].at[wid], src_v)
    pltpu.sync_copy(edges.at[1].at[wid], dst_v)
    pltpu.sync_copy(zrows, shared_agg.at[pl.ds(s * _RPT, _RPT)])
    if with_cnt:
      pltpu.sync_copy(zcnt, shared_cnt.at[pl.ds(s * _RPT, _RPT)])
      pltpu.sync_copy(ones, ones_v)
    plsc.subcore_barrier()

    for b in range(nbuf):
      pltpu.async_copy(feat.at[src_v.at[b]], rows_v.at[b], gsem[b])

    def step(jj, carry):
      for b in range(nbuf):
        j = jj * nbuf + b
        pltpu.make_async_copy(
            feat.at[src_v.at[j]], rows_v.at[b], gsem[b]).wait()
        h = pltpu.async_copy(
            rows_v.at[b], shared_agg.at[dst_v.at[j]], ssem[b], add=True)
        if with_cnt:
          h2 = pltpu.async_copy(
              ones_v, shared_cnt.at[dst_v.at[j]], ssem[b], add=True)
        h.wait()
        if with_cnt:
          h2.wait()

        @pl.when(jj < ch // nbuf - 1)
        def _():
          pltpu.async_copy(
              feat.at[src_v.at[j + nbuf]], rows_v.at[b], gsem[b])
      return carry

    lax.fori_loop(0, ch // nbuf, step, 0)
    plsc.subcore_barrier()

    pltpu.sync_copy(shared_agg.at[pl.ds(s * _RPT, _RPT)],
                    agg_out.at[c].at[pl.ds(s * _RPT, _RPT)])
    if with_cnt:
      pltpu.sync_copy(shared_cnt.at[pl.ds(s * _RPT, _RPT)],
                      cnt_out.at[c].at[pl.ds(s * _RPT, _RPT)])

  return pl.kernel(
      body, out_type=out_type, mesh=mesh, scratch_types=scratch,
      compiler_params=pltpu.CompilerParams(use_tc_tiling_on_sc=False))


def _tc1_body(x_ref, agg_ref, cnt_ref, w1l_ref, b1_ref, w1r_ref,
              w2l_ref, b2_ref, w2r_ref, y2_ref, r2_ref):
  cnt = jnp.maximum(cnt_ref[0, :_N_NODES] + cnt_ref[1, :_N_NODES], 1.0)
  mean = (agg_ref[0, :_N_NODES] + agg_ref[1, :_N_NODES]) / cnt[:, None]
  h = jnp.dot(mean, w1l_ref[...], preferred_element_type=jnp.float32)
  h = h + b1_ref[...]
  h = h + jnp.dot(x_ref[...], w1r_ref[...], preferred_element_type=jnp.float32)
  h = jnp.maximum(h, 0.0)
  y2_ref[...] = jnp.dot(h, w2l_ref[...], preferred_element_type=jnp.float32)
  r2_ref[...] = (jnp.dot(h, w2r_ref[...], preferred_element_type=jnp.float32)
                 + b2_ref[...])


def _tc2_body(agg2_ref, cnt_ref, r2_ref, batch_ref, out_ref):
  cnt = jnp.maximum(cnt_ref[0, :_N_NODES] + cnt_ref[1, :_N_NODES], 1.0)
  h2 = (agg2_ref[0, :_N_NODES] + agg2_ref[1, :_N_NODES]) / cnt[:, None] + r2_ref[...]
  onehot = (lax.broadcasted_iota(jnp.int32, (_N_GRAPHS, _N_NODES), 0)
            == batch_ref[...]).astype(jnp.float32)
  psum = jnp.dot(onehot, h2, preferred_element_type=jnp.float32)
  gcnt = jnp.sum(onehot, axis=1, keepdims=True)
  pooled = psum / jnp.maximum(gcnt, 1.0)
  m = jnp.max(pooled, axis=1, keepdims=True)
  lse = m + jnp.log(jnp.sum(jnp.exp(pooled - m), axis=1, keepdims=True))
  out_ref[...] = pooled - lse


def kernel(x, edge_index, batch, W1_l, b1_l, W1_r, W2_l, b2_l, W2_r):
  e_r = edge_index.reshape(2, _NW, _CH, _BS)
  batch_p = batch.reshape(1, _N_NODES)
  z128 = jnp.zeros((_RPT, _D_IN), jnp.float32)
  z16 = jnp.zeros((_RPT, _D_OUT2), jnp.float32)
  zc = jnp.zeros((_RPT,), jnp.float32)
  ones1 = jnp.ones((_BS,), jnp.float32)

  agg1, cnt = _make_edge_agg(_D_IN, True, _BS, _CH, _NB)(
      x, e_r, z128, zc, ones1)

  y2, r2 = pl.pallas_call(
      _tc1_body,
      out_shape=[jax.ShapeDtypeStruct((_N_NODES, _D_OUT2), jnp.float32),
                 jax.ShapeDtypeStruct((_N_NODES, _D_OUT2), jnp.float32)],
  )(x, agg1, cnt, W1_l, b1_l.reshape(1, -1), W1_r,
    W2_l, b2_l.reshape(1, -1), W2_r)

  (agg2,) = _make_edge_agg(_D_OUT2, False, _BS, _CH, _NB)(
      y2, e_r, z16, zc, ones1)

  out = pl.pallas_call(
      _tc2_body,
      out_shape=jax.ShapeDtypeStruct((_N_GRAPHS, _D_OUT2), jnp.float32),
  )(agg2, cnt, r2, batch_p)
  return out
